# baseline (speedup 1.0000x reference)
import functools

import jax
import jax.numpy as jnp
from jax import lax
from jax.experimental import pallas as pl
from jax.experimental.pallas import tpu as pltpu
from jax.experimental.pallas import tpu_sc as plsc

N_TILES = 16
CHUNK = 128


PAD_ROWS = 16


def _make_segsum(n, feat, chunks_per_tile):
    cpt = chunks_per_tile
    seg = 32
    assert cpt % seg == 0
    nseg = cpt // seg
    rpt = (n // N_TILES) // 8 * 8
    tail0 = N_TILES * rpt
    tail = n - tail0

    mesh = plsc.VectorSubcoreMesh(core_axis_name="c", subcore_axis_name="s")

    @functools.partial(
        pl.kernel,
        mesh=mesh,
        out_type=jax.ShapeDtypeStruct((2, n, feat), jnp.float32),
        scratch_types=[
            pltpu.VMEM((seg, CHUNK), jnp.int32),
            pltpu.VMEM((seg, CHUNK), jnp.int32),
            pltpu.VMEM((2, CHUNK, feat), jnp.float32),
            pltpu.VMEM_SHARED((n + PAD_ROWS, feat), jnp.float32),
            pltpu.SemaphoreType.DMA,
            pltpu.SemaphoreType.DMA,
        ],
    )
    def segsum(h_hbm, srcs_hbm, dsts_hbm, out_hbm, src2d, dst2d, rows, acc,
               sem, sem_s):
        c = lax.axis_index("c")
        s = lax.axis_index("s")
        ch0 = (c * N_TILES + s) * cpt

        row0 = s * rpt
        pltpu.sync_copy(h_hbm.at[pl.ds(row0, rpt)], acc.at[pl.ds(row0, rpt)])
        if tail:
            @pl.when(s == 0)
            def _():
                pltpu.sync_copy(h_hbm.at[pl.ds(tail0, tail)],
                                acc.at[pl.ds(tail0, tail)])
        plsc.subcore_barrier()

        def wait_gather(p):
            pltpu.make_async_copy(h_hbm.at[src2d.at[0]], rows.at[p], sem).wait()

        def wait_scatter(p):
            pltpu.make_async_copy(rows.at[p], acc.at[dst2d.at[0]], sem_s).wait()

        def seg_body(g, carry):
            pltpu.sync_copy(srcs_hbm.at[pl.ds(ch0 + g * seg, seg)], src2d)
            pltpu.sync_copy(dsts_hbm.at[pl.ds(ch0 + g * seg, seg)], dst2d)

            pltpu.async_copy(h_hbm.at[src2d.at[0]], rows.at[0], sem)

            def body(i, carry2):
                p = lax.rem(i, 2)
                wait_gather(p)
                @pl.when(i >= 1)
                def _():
                    wait_scatter(p)
                @pl.when(i + 1 < seg)
                def _():
                    pltpu.async_copy(h_hbm.at[src2d.at[i + 1]], rows.at[1 - p],
                                     sem)
                pltpu.async_copy(rows.at[p], acc.at[dst2d.at[i]], sem_s,
                                 add=True)
                return carry2

            lax.fori_loop(0, seg, body, carry)
            wait_scatter(0)
            return carry

        lax.fori_loop(0, nseg, seg_body, 0)
        plsc.subcore_barrier()

        pltpu.sync_copy(acc.at[pl.ds(row0, rpt)],
                        out_hbm.at[c, pl.ds(row0, rpt)])
        if tail:
            @pl.when(s == 0)
            def _():
                pltpu.sync_copy(acc.at[pl.ds(tail0, tail)],
                                out_hbm.at[c, pl.ds(tail0, tail)])

    return segsum


def _dense_body(a0_ref, a1_ref, w10, b10, w20, b20, w11, b11, w21, b21,
                sa_w, sa_b, sa_q, out_ref):
    def gin_mlp(a, w1, b1, w2, b2):
        h = jnp.maximum(
            jnp.dot(a, w1[...], preferred_element_type=jnp.float32) + b1[...], 0.0)
        t = jnp.dot(h, w2[...], preferred_element_type=jnp.float32) + b2[...]
        return jnp.where(t > 0, t, jnp.exp(jnp.minimum(t, 0.0)) - 1.0)

    e0 = gin_mlp(a0_ref[...], w10, b10, w20, b20)
    e1 = gin_mlp(a1_ref[...], w11, b11, w21, b21)

    def score(e):
        w = jnp.tanh(jnp.dot(e, sa_w[...], preferred_element_type=jnp.float32)
                     + sa_b[...])
        return jnp.dot(w, sa_q[...], preferred_element_type=jnp.float32)

    s0 = score(e0)
    s1 = score(e1)
    m = jnp.maximum(s0, s1)
    x0 = jnp.exp(s0 - m)
    x1 = jnp.exp(s1 - m)
    inv = 1.0 / (x0 + x1)
    out_ref[...] = (x0 * inv) * e0 + (x1 * inv) * e1


def _make_dense(n, feat, dim_a, block_rows=1000):
    assert n % block_rows == 0
    grid = n // block_rows
    row_spec = pl.BlockSpec((block_rows, feat), lambda i: (i, 0))
    full = lambda shape: pl.BlockSpec(shape, lambda i: (0,) * len(shape))
    return pl.pallas_call(
        _dense_body,
        grid=(grid,),
        in_specs=[
            row_spec, row_spec,
            full((feat, feat)), full((1, feat)), full((feat, feat)), full((1, feat)),
            full((feat, feat)), full((1, feat)), full((feat, feat)), full((1, feat)),
            full((feat, dim_a)), full((1, dim_a)), full((dim_a, 1)),
        ],
        out_specs=row_spec,
        out_shape=jax.ShapeDtypeStruct((n, feat), jnp.float32),
    )


def kernel(x, edge_index_r0, edge_index_r1,
           l0_r0_W1, l0_r0_b1, l0_r0_W2, l0_r0_b2,
           l0_r1_W1, l0_r1_b1, l0_r1_W2, l0_r1_b2,
           l1_r0_W1, l1_r0_b1, l1_r0_W2, l1_r0_b2,
           l1_r1_W1, l1_r1_b1, l1_r1_W2, l1_r1_b2,
           sa_W, sa_b, sa_q):
    n, feat = x.shape
    e = edge_index_r0.shape[1]
    dim_a = sa_W.shape[1]

    grain = N_TILES * CHUNK * 8
    e_pad = -(-e // grain) * grain
    npad = e_pad - e
    cpt = e_pad // (N_TILES * CHUNK)
    pad_src = (jnp.arange(npad, dtype=jnp.int32) * 613) % n
    pad_dst = n + jnp.arange(npad, dtype=jnp.int32) % PAD_ROWS
    srcs = jnp.concatenate([edge_index_r0[0], pad_src,
                            edge_index_r1[0], pad_src]).reshape(-1, CHUNK)
    dsts = jnp.concatenate([edge_index_r0[1], pad_dst,
                            edge_index_r1[1], pad_dst]).reshape(-1, CHUNK)

    segsum = _make_segsum(n, feat, cpt)
    dense = _make_dense(n, feat, dim_a)

    def layer(h, params):
        (w10, b10, w20, b20), (w11, b11, w21, b21) = params
        agg = segsum(h, srcs, dsts)
        return dense(agg[0], agg[1],
                     w10, b10.reshape(1, feat), w20, b20.reshape(1, feat),
                     w11, b11.reshape(1, feat), w21, b21.reshape(1, feat),
                     sa_W, sa_b, sa_q)

    h = layer(x, ((l0_r0_W1, l0_r0_b1, l0_r0_W2, l0_r0_b2),
                  (l0_r1_W1, l0_r1_b1, l0_r1_W2, l0_r1_b2)))
    h = layer(h, ((l1_r0_W1, l1_r0_b1, l1_r0_W2, l1_r0_b2),
                  (l1_r1_W1, l1_r1_b1, l1_r1_W2, l1_r1_b2)))
    return h


# speedup vs baseline: 1.2385x; 1.2385x over previous
import functools

import jax
import jax.numpy as jnp
from jax import lax
from jax.experimental import pallas as pl
from jax.experimental.pallas import tpu as pltpu
from jax.experimental.pallas import tpu_sc as plsc

N_TILES = 16
CHUNK = 128
SEG = 32
PAD_ROWS = 16


def _make_segsum(n, feat, total_chunks):
    cpt = total_chunks // N_TILES
    assert cpt * N_TILES == total_chunks and cpt % SEG == 0
    nseg = cpt // SEG
    rel_stride = total_chunks
    rpt = (n // N_TILES) // 8 * 8
    tail0 = N_TILES * rpt
    tail = n - tail0

    mesh = plsc.VectorSubcoreMesh(core_axis_name="c", subcore_axis_name="s")

    @functools.partial(
        pl.kernel,
        mesh=mesh,
        out_type=jax.ShapeDtypeStruct((2, n, feat), jnp.float32),
        scratch_types=[
            pltpu.VMEM((SEG, CHUNK), jnp.int32),
            pltpu.VMEM((SEG, CHUNK), jnp.int32),
            pltpu.VMEM((SEG, CHUNK), jnp.int32),
            pltpu.VMEM((SEG, CHUNK), jnp.int32),
            pltpu.VMEM((2, CHUNK, feat), jnp.float32),
            pltpu.VMEM_SHARED((n + PAD_ROWS, feat), jnp.float32),
            pltpu.SemaphoreType.DMA,
            pltpu.SemaphoreType.DMA,
        ],
    )
    def segsum(h_hbm, srcs_hbm, dsts_hbm, out_hbm,
               src_a, dst_a, src_b, dst_b, rows, acc, sem, sem_i):
        c = lax.axis_index("c")
        s = lax.axis_index("s")
        ch0 = c * rel_stride + s * cpt

        row0 = s * rpt
        pltpu.sync_copy(h_hbm.at[pl.ds(row0, rpt)], acc.at[pl.ds(row0, rpt)])
        if tail:
            @pl.when(s == 0)
            def _():
                pltpu.sync_copy(h_hbm.at[pl.ds(tail0, tail)],
                                acc.at[pl.ds(tail0, tail)])
        plsc.subcore_barrier()

        bufs = [(src_a, dst_a), (src_b, dst_b)]
        pltpu.sync_copy(srcs_hbm.at[pl.ds(ch0, SEG)], src_a)
        pltpu.sync_copy(dsts_hbm.at[pl.ds(ch0, SEG)], dst_a)

        for g in range(nseg):
            src2d, dst2d = bufs[g % 2]
            nsrc2d, ndst2d = bufs[(g + 1) % 2]
            if g + 1 < nseg:
                pltpu.async_copy(srcs_hbm.at[pl.ds(ch0 + (g + 1) * SEG, SEG)],
                                 nsrc2d, sem_i)
                pltpu.async_copy(dsts_hbm.at[pl.ds(ch0 + (g + 1) * SEG, SEG)],
                                 ndst2d, sem_i)

            pltpu.async_copy(h_hbm.at[src2d.at[0]], rows.at[0], sem)

            def body(i, carry, src2d=src2d, dst2d=dst2d):
                p = lax.rem(i, 2)
                @pl.when(i + 1 < SEG)
                def _():
                    pltpu.async_copy(h_hbm.at[src2d.at[i + 1]], rows.at[1 - p],
                                     sem)
                pltpu.make_async_copy(h_hbm.at[src2d.at[0]], rows.at[p],
                                      sem).wait()
                pltpu.sync_copy(rows.at[p], acc.at[dst2d.at[i]], add=True)
                return carry

            lax.fori_loop(0, SEG, body, 0)
            if g + 1 < nseg:
                pltpu.make_async_copy(srcs_hbm.at[pl.ds(0, SEG)], nsrc2d,
                                      sem_i).wait()
                pltpu.make_async_copy(dsts_hbm.at[pl.ds(0, SEG)], ndst2d,
                                      sem_i).wait()

        plsc.subcore_barrier()

        pltpu.sync_copy(acc.at[pl.ds(row0, rpt)],
                        out_hbm.at[c, pl.ds(row0, rpt)])
        if tail:
            @pl.when(s == 0)
            def _():
                pltpu.sync_copy(acc.at[pl.ds(tail0, tail)],
                                out_hbm.at[c, pl.ds(tail0, tail)])

    return segsum


def _dense_body(a0_ref, a1_ref, w10, b10, w20, b20, w11, b11, w21, b21,
                sa_w, sa_b, sa_q, out_ref):
    def gin_mlp(a, w1, b1, w2, b2):
        h = jnp.maximum(
            jnp.dot(a, w1[...], preferred_element_type=jnp.float32) + b1[...], 0.0)
        t = jnp.dot(h, w2[...], preferred_element_type=jnp.float32) + b2[...]
        return jnp.where(t > 0, t, jnp.exp(jnp.minimum(t, 0.0)) - 1.0)

    e0 = gin_mlp(a0_ref[...], w10, b10, w20, b20)
    e1 = gin_mlp(a1_ref[...], w11, b11, w21, b21)

    def score(e):
        w = jnp.tanh(jnp.dot(e, sa_w[...], preferred_element_type=jnp.float32)
                     + sa_b[...])
        return jnp.dot(w, sa_q[...], preferred_element_type=jnp.float32)

    s0 = score(e0)
    s1 = score(e1)
    m = jnp.maximum(s0, s1)
    x0 = jnp.exp(s0 - m)
    x1 = jnp.exp(s1 - m)
    inv = 1.0 / (x0 + x1)
    out_ref[...] = (x0 * inv) * e0 + (x1 * inv) * e1


def _make_dense(n, feat, dim_a, block_rows=2000):
    assert n % block_rows == 0
    grid = n // block_rows
    row_spec = pl.BlockSpec((block_rows, feat), lambda i: (i, 0))
    full = lambda shape: pl.BlockSpec(shape, lambda i: (0,) * len(shape))
    return pl.pallas_call(
        _dense_body,
        grid=(grid,),
        in_specs=[
            row_spec, row_spec,
            full((feat, feat)), full((1, feat)), full((feat, feat)), full((1, feat)),
            full((feat, feat)), full((1, feat)), full((feat, feat)), full((1, feat)),
            full((feat, dim_a)), full((1, dim_a)), full((dim_a, 1)),
        ],
        out_specs=row_spec,
        out_shape=jax.ShapeDtypeStruct((n, feat), jnp.float32),
    )


def kernel(x, edge_index_r0, edge_index_r1,
           l0_r0_W1, l0_r0_b1, l0_r0_W2, l0_r0_b2,
           l0_r1_W1, l0_r1_b1, l0_r1_W2, l0_r1_b2,
           l1_r0_W1, l1_r0_b1, l1_r0_W2, l1_r0_b2,
           l1_r1_W1, l1_r1_b1, l1_r1_W2, l1_r1_b2,
           sa_W, sa_b, sa_q):
    n, feat = x.shape
    e = edge_index_r0.shape[1]
    dim_a = sa_W.shape[1]

    grain = N_TILES * CHUNK * SEG
    e_pad = -(-e // grain) * grain
    npad = e_pad - e
    pad_src = (jnp.arange(npad, dtype=jnp.int32) * 613) % n
    pad_dst = n + jnp.arange(npad, dtype=jnp.int32) % PAD_ROWS
    srcs = jnp.concatenate([edge_index_r0[0], pad_src,
                            edge_index_r1[0], pad_src]).reshape(-1, CHUNK)
    dsts = jnp.concatenate([edge_index_r0[1], pad_dst,
                            edge_index_r1[1], pad_dst]).reshape(-1, CHUNK)

    segsum = _make_segsum(n, feat, e_pad // CHUNK)
    dense = _make_dense(n, feat, dim_a)

    def layer(h, params):
        (w10, b10, w20, b20), (w11, b11, w21, b21) = params
        agg = segsum(h, srcs, dsts)
        return dense(agg[0], agg[1],
                     w10, b10.reshape(1, feat), w20, b20.reshape(1, feat),
                     w11, b11.reshape(1, feat), w21, b21.reshape(1, feat),
                     sa_W, sa_b, sa_q)

    h = layer(x, ((l0_r0_W1, l0_r0_b1, l0_r0_W2, l0_r0_b2),
                  (l0_r1_W1, l0_r1_b1, l0_r1_W2, l0_r1_b2)))
    h = layer(h, ((l1_r0_W1, l1_r0_b1, l1_r0_W2, l1_r0_b2),
                  (l1_r1_W1, l1_r1_b1, l1_r1_W2, l1_r1_b2)))
    return h


# speedup vs baseline: 1.2636x; 1.0203x over previous
import functools

import jax
import jax.numpy as jnp
from jax import lax
from jax.experimental import pallas as pl
from jax.experimental.pallas import tpu as pltpu
from jax.experimental.pallas import tpu_sc as plsc

N_TILES = 16
CHUNK = 128
SEG = 32
PAD_ROWS = 16


def _make_segsum(n, feat, total_chunks):
    cpt = total_chunks // N_TILES
    assert cpt * N_TILES == total_chunks and cpt % SEG == 0
    nseg = cpt // SEG
    rel_stride = total_chunks
    rpt = (n // N_TILES) // 8 * 8
    tail0 = N_TILES * rpt
    tail = n - tail0

    mesh = plsc.VectorSubcoreMesh(core_axis_name="c", subcore_axis_name="s")

    @functools.partial(
        pl.kernel,
        mesh=mesh,
        out_type=jax.ShapeDtypeStruct((2, n, feat), jnp.float32),
        scratch_types=[
            pltpu.VMEM((SEG, CHUNK), jnp.int32),
            pltpu.VMEM((SEG, CHUNK), jnp.int32),
            pltpu.VMEM((SEG, CHUNK), jnp.int32),
            pltpu.VMEM((SEG, CHUNK), jnp.int32),
            pltpu.VMEM((2, CHUNK, feat), jnp.float32),
            pltpu.VMEM_SHARED((n + PAD_ROWS, feat), jnp.float32),
            pltpu.SemaphoreType.DMA,
            pltpu.SemaphoreType.DMA,
        ],
    )
    def segsum(h_hbm, srcs_hbm, dsts_hbm, out_hbm,
               src_a, dst_a, src_b, dst_b, rows, acc, sem, sem_i):
        c = lax.axis_index("c")
        s = lax.axis_index("s")
        ch0 = c * rel_stride + s * cpt

        row0 = s * rpt
        pltpu.sync_copy(h_hbm.at[pl.ds(row0, rpt)], acc.at[pl.ds(row0, rpt)])
        if tail:
            @pl.when(s == 0)
            def _():
                pltpu.sync_copy(h_hbm.at[pl.ds(tail0, tail)],
                                acc.at[pl.ds(tail0, tail)])
        plsc.subcore_barrier()

        bufs = [(src_a, dst_a), (src_b, dst_b)]
        pltpu.sync_copy(srcs_hbm.at[pl.ds(ch0, SEG)], src_a)
        pltpu.sync_copy(dsts_hbm.at[pl.ds(ch0, SEG)], dst_a)

        for g in range(nseg):
            src2d, dst2d = bufs[g % 2]
            nsrc2d, ndst2d = bufs[(g + 1) % 2]
            if g + 1 < nseg:
                pltpu.async_copy(srcs_hbm.at[pl.ds(ch0 + (g + 1) * SEG, SEG)],
                                 nsrc2d, sem_i)
                pltpu.async_copy(dsts_hbm.at[pl.ds(ch0 + (g + 1) * SEG, SEG)],
                                 ndst2d, sem_i)

            pltpu.async_copy(h_hbm.at[src2d.at[0]], rows.at[0], sem)

            def body(i, carry, src2d=src2d, dst2d=dst2d):
                p = lax.rem(i, 2)
                @pl.when(i + 1 < SEG)
                def _():
                    pltpu.async_copy(h_hbm.at[src2d.at[i + 1]], rows.at[1 - p],
                                     sem)
                pltpu.make_async_copy(h_hbm.at[src2d.at[0]], rows.at[p],
                                      sem).wait()
                pltpu.sync_copy(rows.at[p], acc.at[dst2d.at[i]], add=True)
                return carry

            lax.fori_loop(0, SEG, body, 0)
            if g + 1 < nseg:
                pltpu.make_async_copy(srcs_hbm.at[pl.ds(0, SEG)], nsrc2d,
                                      sem_i).wait()
                pltpu.make_async_copy(dsts_hbm.at[pl.ds(0, SEG)], ndst2d,
                                      sem_i).wait()

        plsc.subcore_barrier()

        pltpu.sync_copy(acc.at[pl.ds(row0, rpt)],
                        out_hbm.at[c, pl.ds(row0, rpt)])
        if tail:
            @pl.when(s == 0)
            def _():
                pltpu.sync_copy(acc.at[pl.ds(tail0, tail)],
                                out_hbm.at[c, pl.ds(tail0, tail)])

    return segsum


def _dense_body(agg0_ref, agg1_ref, w10, b10, w20, b20, w11, b11, w21, b21,
                sa_w, sa_b, sa_q, out_ref):
    a0_ref = agg0_ref.at[0]
    a1_ref = agg1_ref.at[0]
    def gin_mlp(a, w1, b1, w2, b2):
        h = jnp.maximum(
            jnp.dot(a, w1[...], preferred_element_type=jnp.float32) + b1[...], 0.0)
        t = jnp.dot(h, w2[...], preferred_element_type=jnp.float32) + b2[...]
        return jnp.where(t > 0, t, jnp.exp(jnp.minimum(t, 0.0)) - 1.0)

    e0 = gin_mlp(a0_ref[...], w10, b10, w20, b20)
    e1 = gin_mlp(a1_ref[...], w11, b11, w21, b21)

    def score(e):
        w = jnp.tanh(jnp.dot(e, sa_w[...], preferred_element_type=jnp.float32)
                     + sa_b[...])
        return jnp.dot(w, sa_q[...], preferred_element_type=jnp.float32)

    s0 = score(e0)
    s1 = score(e1)
    m = jnp.maximum(s0, s1)
    x0 = jnp.exp(s0 - m)
    x1 = jnp.exp(s1 - m)
    inv = 1.0 / (x0 + x1)
    out_ref[...] = (x0 * inv) * e0 + (x1 * inv) * e1


def _make_dense(n, feat, dim_a, block_rows=2000):
    assert n % block_rows == 0
    grid = n // block_rows
    row_spec = pl.BlockSpec((block_rows, feat), lambda i: (i, 0))
    agg_spec = lambda r: pl.BlockSpec((1, block_rows, feat), lambda i: (r, i, 0))
    full = lambda shape: pl.BlockSpec(shape, lambda i: (0,) * len(shape))
    return pl.pallas_call(
        _dense_body,
        grid=(grid,),
        in_specs=[
            agg_spec(0), agg_spec(1),
            full((feat, feat)), full((1, feat)), full((feat, feat)), full((1, feat)),
            full((feat, feat)), full((1, feat)), full((feat, feat)), full((1, feat)),
            full((feat, dim_a)), full((1, dim_a)), full((dim_a, 1)),
        ],
        out_specs=row_spec,
        out_shape=jax.ShapeDtypeStruct((n, feat), jnp.float32),
    )


def kernel(x, edge_index_r0, edge_index_r1,
           l0_r0_W1, l0_r0_b1, l0_r0_W2, l0_r0_b2,
           l0_r1_W1, l0_r1_b1, l0_r1_W2, l0_r1_b2,
           l1_r0_W1, l1_r0_b1, l1_r0_W2, l1_r0_b2,
           l1_r1_W1, l1_r1_b1, l1_r1_W2, l1_r1_b2,
           sa_W, sa_b, sa_q):
    n, feat = x.shape
    e = edge_index_r0.shape[1]
    dim_a = sa_W.shape[1]

    grain = N_TILES * CHUNK * SEG
    e_pad = -(-e // grain) * grain
    npad = e_pad - e
    pad_src = (jnp.arange(npad, dtype=jnp.int32) * 613) % n
    pad_dst = n + jnp.arange(npad, dtype=jnp.int32) % PAD_ROWS
    srcs = jnp.concatenate([edge_index_r0[0], pad_src,
                            edge_index_r1[0], pad_src]).reshape(-1, CHUNK)
    dsts = jnp.concatenate([edge_index_r0[1], pad_dst,
                            edge_index_r1[1], pad_dst]).reshape(-1, CHUNK)

    segsum = _make_segsum(n, feat, e_pad // CHUNK)
    dense = _make_dense(n, feat, dim_a)

    def layer(h, params):
        (w10, b10, w20, b20), (w11, b11, w21, b21) = params
        agg = segsum(h, srcs, dsts)
        return dense(agg, agg,
                     w10, b10.reshape(1, feat), w20, b20.reshape(1, feat),
                     w11, b11.reshape(1, feat), w21, b21.reshape(1, feat),
                     sa_W, sa_b, sa_q)

    h = layer(x, ((l0_r0_W1, l0_r0_b1, l0_r0_W2, l0_r0_b2),
                  (l0_r1_W1, l0_r1_b1, l0_r1_W2, l0_r1_b2)))
    h = layer(h, ((l1_r0_W1, l1_r0_b1, l1_r0_W2, l1_r0_b2),
                  (l1_r1_W1, l1_r1_b1, l1_r1_W2, l1_r1_b2)))
    return h


# speedup vs baseline: 1.2932x; 1.0234x over previous
import functools

import jax
import jax.numpy as jnp
from jax import lax
from jax.experimental import pallas as pl
from jax.experimental.pallas import tpu as pltpu
from jax.experimental.pallas import tpu_sc as plsc

N_TILES = 16
CHUNK = 128
SEG = 32
PAD_ROWS = 16


def _make_segsum(n, feat, total_chunks):
    cpt = total_chunks // N_TILES
    assert cpt * N_TILES == total_chunks and cpt % SEG == 0
    nseg = cpt // SEG
    rel_stride = total_chunks
    rpt = (n // N_TILES) // 8 * 8
    tail0 = N_TILES * rpt
    tail = n - tail0

    mesh = plsc.VectorSubcoreMesh(core_axis_name="c", subcore_axis_name="s")

    @functools.partial(
        pl.kernel,
        mesh=mesh,
        out_type=jax.ShapeDtypeStruct((2, n, feat), jnp.float32),
        scratch_types=[
            pltpu.VMEM((SEG, CHUNK), jnp.int32),
            pltpu.VMEM((SEG, CHUNK), jnp.int32),
            pltpu.VMEM((SEG, CHUNK), jnp.int32),
            pltpu.VMEM((SEG, CHUNK), jnp.int32),
            pltpu.VMEM((2, CHUNK, feat), jnp.float32),
            pltpu.VMEM_SHARED((n + PAD_ROWS, feat), jnp.float32),
            pltpu.SemaphoreType.DMA,
            pltpu.SemaphoreType.DMA,
        ],
    )
    def segsum(h_hbm, srcs_hbm, dsts_hbm, out_hbm,
               src_a, dst_a, src_b, dst_b, rows, acc, sem, sem_i):
        c = lax.axis_index("c")
        s = lax.axis_index("s")
        ch0 = c * rel_stride + s * cpt

        pltpu.async_copy(srcs_hbm.at[pl.ds(ch0, SEG)], src_a, sem_i)
        pltpu.async_copy(dsts_hbm.at[pl.ds(ch0, SEG)], dst_a, sem_i)

        row0 = s * rpt
        pltpu.sync_copy(h_hbm.at[pl.ds(row0, rpt)], acc.at[pl.ds(row0, rpt)])
        if tail:
            @pl.when(s == 0)
            def _():
                pltpu.sync_copy(h_hbm.at[pl.ds(tail0, tail)],
                                acc.at[pl.ds(tail0, tail)])
        plsc.subcore_barrier()

        bufs = [(src_a, dst_a), (src_b, dst_b)]

        def wait_idx(sref, dref):
            pltpu.make_async_copy(srcs_hbm.at[pl.ds(0, SEG)], sref,
                                  sem_i).wait()
            pltpu.make_async_copy(dsts_hbm.at[pl.ds(0, SEG)], dref,
                                  sem_i).wait()

        wait_idx(src_a, dst_a)
        pltpu.async_copy(h_hbm.at[src_a.at[0]], rows.at[0], sem)

        assert SEG % 2 == 0
        for g in range(nseg):
            src2d, dst2d = bufs[g % 2]
            nsrc2d, ndst2d = bufs[(g + 1) % 2]
            if g + 1 < nseg:
                pltpu.async_copy(srcs_hbm.at[pl.ds(ch0 + (g + 1) * SEG, SEG)],
                                 nsrc2d, sem_i)
                pltpu.async_copy(dsts_hbm.at[pl.ds(ch0 + (g + 1) * SEG, SEG)],
                                 ndst2d, sem_i)

            def body(i, carry, src2d=src2d, dst2d=dst2d):
                p = lax.rem(i, 2)
                pltpu.async_copy(h_hbm.at[src2d.at[i + 1]], rows.at[1 - p], sem)
                pltpu.make_async_copy(h_hbm.at[src2d.at[0]], rows.at[p],
                                      sem).wait()
                pltpu.sync_copy(rows.at[p], acc.at[dst2d.at[i]], add=True)
                return carry

            lax.fori_loop(0, SEG - 1, body, 0)

            if g + 1 < nseg:
                wait_idx(nsrc2d, ndst2d)
                pltpu.async_copy(h_hbm.at[nsrc2d.at[0]], rows.at[0], sem)
            pltpu.make_async_copy(h_hbm.at[src2d.at[0]], rows.at[1], sem).wait()
            pltpu.sync_copy(rows.at[1], acc.at[dst2d.at[SEG - 1]], add=True)

        plsc.subcore_barrier()

        pltpu.sync_copy(acc.at[pl.ds(row0, rpt)],
                        out_hbm.at[c, pl.ds(row0, rpt)])
        if tail:
            @pl.when(s == 0)
            def _():
                pltpu.sync_copy(acc.at[pl.ds(tail0, tail)],
                                out_hbm.at[c, pl.ds(tail0, tail)])

    return segsum


def _dense_body(agg0_ref, agg1_ref, w10, b10, w20, b20, w11, b11, w21, b21,
                sa_w, sa_b, sa_q, out_ref):
    a0_ref = agg0_ref.at[0]
    a1_ref = agg1_ref.at[0]
    def gin_mlp(a, w1, b1, w2, b2):
        h = jnp.maximum(
            jnp.dot(a, w1[...], preferred_element_type=jnp.float32) + b1[...], 0.0)
        t = jnp.dot(h, w2[...], preferred_element_type=jnp.float32) + b2[...]
        return jnp.where(t > 0, t, jnp.exp(jnp.minimum(t, 0.0)) - 1.0)

    e0 = gin_mlp(a0_ref[...], w10, b10, w20, b20)
    e1 = gin_mlp(a1_ref[...], w11, b11, w21, b21)

    def score(e):
        w = jnp.tanh(jnp.dot(e, sa_w[...], preferred_element_type=jnp.float32)
                     + sa_b[...])
        return jnp.dot(w, sa_q[...], preferred_element_type=jnp.float32)

    s0 = score(e0)
    s1 = score(e1)
    m = jnp.maximum(s0, s1)
    x0 = jnp.exp(s0 - m)
    x1 = jnp.exp(s1 - m)
    inv = 1.0 / (x0 + x1)
    out_ref[...] = (x0 * inv) * e0 + (x1 * inv) * e1


def _make_dense(n, feat, dim_a, block_rows=2000):
    assert n % block_rows == 0
    grid = n // block_rows
    row_spec = pl.BlockSpec((block_rows, feat), lambda i: (i, 0))
    agg_spec = lambda r: pl.BlockSpec((1, block_rows, feat), lambda i: (r, i, 0))
    full = lambda shape: pl.BlockSpec(shape, lambda i: (0,) * len(shape))
    return pl.pallas_call(
        _dense_body,
        grid=(grid,),
        in_specs=[
            agg_spec(0), agg_spec(1),
            full((feat, feat)), full((1, feat)), full((feat, feat)), full((1, feat)),
            full((feat, feat)), full((1, feat)), full((feat, feat)), full((1, feat)),
            full((feat, dim_a)), full((1, dim_a)), full((dim_a, 1)),
        ],
        out_specs=row_spec,
        out_shape=jax.ShapeDtypeStruct((n, feat), jnp.float32),
    )


def kernel(x, edge_index_r0, edge_index_r1,
           l0_r0_W1, l0_r0_b1, l0_r0_W2, l0_r0_b2,
           l0_r1_W1, l0_r1_b1, l0_r1_W2, l0_r1_b2,
           l1_r0_W1, l1_r0_b1, l1_r0_W2, l1_r0_b2,
           l1_r1_W1, l1_r1_b1, l1_r1_W2, l1_r1_b2,
           sa_W, sa_b, sa_q):
    n, feat = x.shape
    e = edge_index_r0.shape[1]
    dim_a = sa_W.shape[1]

    grain = N_TILES * CHUNK * SEG
    e_pad = -(-e // grain) * grain
    npad = e_pad - e
    pad_src = (jnp.arange(npad, dtype=jnp.int32) * 613) % n
    pad_dst = n + jnp.arange(npad, dtype=jnp.int32) % PAD_ROWS
    srcs = jnp.concatenate([edge_index_r0[0], pad_src,
                            edge_index_r1[0], pad_src]).reshape(-1, CHUNK)
    dsts = jnp.concatenate([edge_index_r0[1], pad_dst,
                            edge_index_r1[1], pad_dst]).reshape(-1, CHUNK)

    segsum = _make_segsum(n, feat, e_pad // CHUNK)
    dense = _make_dense(n, feat, dim_a)

    def layer(h, params):
        (w10, b10, w20, b20), (w11, b11, w21, b21) = params
        agg = segsum(h, srcs, dsts)
        return dense(agg, agg,
                     w10, b10.reshape(1, feat), w20, b20.reshape(1, feat),
                     w11, b11.reshape(1, feat), w21, b21.reshape(1, feat),
                     sa_W, sa_b, sa_q)

    h = layer(x, ((l0_r0_W1, l0_r0_b1, l0_r0_W2, l0_r0_b2),
                  (l0_r1_W1, l0_r1_b1, l0_r1_W2, l0_r1_b2)))
    h = layer(h, ((l1_r0_W1, l1_r0_b1, l1_r0_W2, l1_r0_b2),
                  (l1_r1_W1, l1_r1_b1, l1_r1_W2, l1_r1_b2)))
    return h


# speedup vs baseline: 1.2937x; 1.0003x over previous
import functools

import jax
import jax.numpy as jnp
from jax import lax
from jax.experimental import pallas as pl
from jax.experimental.pallas import tpu as pltpu
from jax.experimental.pallas import tpu_sc as plsc

N_TILES = 16
CHUNK = 128
SEG = 32
PAD_ROWS = 128


def _make_segsum(n, feat, total_chunks):
    cpt = total_chunks // N_TILES
    assert cpt * N_TILES == total_chunks and cpt % SEG == 0
    nseg = cpt // SEG
    rel_stride = total_chunks
    rpt = (n // N_TILES) // 8 * 8
    tail0 = N_TILES * rpt
    tail = n - tail0

    mesh = plsc.VectorSubcoreMesh(core_axis_name="c", subcore_axis_name="s")

    @functools.partial(
        pl.kernel,
        mesh=mesh,
        out_type=jax.ShapeDtypeStruct((2, n, feat), jnp.float32),
        scratch_types=[
            pltpu.VMEM((SEG, CHUNK), jnp.int32),
            pltpu.VMEM((SEG, CHUNK), jnp.int32),
            pltpu.VMEM((SEG, CHUNK), jnp.int32),
            pltpu.VMEM((SEG, CHUNK), jnp.int32),
            pltpu.VMEM((2, CHUNK, feat), jnp.float32),
            pltpu.VMEM_SHARED((n + PAD_ROWS, feat), jnp.float32),
            pltpu.SemaphoreType.DMA,
            pltpu.SemaphoreType.DMA,
        ],
    )
    def segsum(h_hbm, srcs_hbm, dsts_hbm, out_hbm,
               src_a, dst_a, src_b, dst_b, rows, acc, sem, sem_i):
        c = lax.axis_index("c")
        s = lax.axis_index("s")
        ch0 = c * rel_stride + s * cpt

        pltpu.async_copy(srcs_hbm.at[pl.ds(ch0, SEG)], src_a, sem_i)
        pltpu.async_copy(dsts_hbm.at[pl.ds(ch0, SEG)], dst_a, sem_i)

        row0 = s * rpt
        pltpu.sync_copy(h_hbm.at[pl.ds(row0, rpt)], acc.at[pl.ds(row0, rpt)])
        if tail:
            @pl.when(s == 0)
            def _():
                pltpu.sync_copy(h_hbm.at[pl.ds(tail0, tail)],
                                acc.at[pl.ds(tail0, tail)])
        plsc.subcore_barrier()

        bufs = [(src_a, dst_a), (src_b, dst_b)]

        def wait_idx(sref, dref):
            pltpu.make_async_copy(srcs_hbm.at[pl.ds(0, SEG)], sref,
                                  sem_i).wait()
            pltpu.make_async_copy(dsts_hbm.at[pl.ds(0, SEG)], dref,
                                  sem_i).wait()

        wait_idx(src_a, dst_a)
        pltpu.async_copy(h_hbm.at[src_a.at[0]], rows.at[0], sem)

        assert SEG % 2 == 0
        for g in range(nseg):
            src2d, dst2d = bufs[g % 2]
            nsrc2d, ndst2d = bufs[(g + 1) % 2]
            if g + 1 < nseg:
                pltpu.async_copy(srcs_hbm.at[pl.ds(ch0 + (g + 1) * SEG, SEG)],
                                 nsrc2d, sem_i)
                pltpu.async_copy(dsts_hbm.at[pl.ds(ch0 + (g + 1) * SEG, SEG)],
                                 ndst2d, sem_i)

            def body(i, carry, src2d=src2d, dst2d=dst2d):
                p = lax.rem(i, 2)
                pltpu.async_copy(h_hbm.at[src2d.at[i + 1]], rows.at[1 - p], sem)
                pltpu.make_async_copy(h_hbm.at[src2d.at[0]], rows.at[p],
                                      sem).wait()
                pltpu.sync_copy(rows.at[p], acc.at[dst2d.at[i]], add=True)
                return carry

            lax.fori_loop(0, SEG - 1, body, 0, unroll=2)

            if g + 1 < nseg:
                wait_idx(nsrc2d, ndst2d)
                pltpu.async_copy(h_hbm.at[nsrc2d.at[0]], rows.at[0], sem)
            pltpu.make_async_copy(h_hbm.at[src2d.at[0]], rows.at[1], sem).wait()
            pltpu.sync_copy(rows.at[1], acc.at[dst2d.at[SEG - 1]], add=True)

        plsc.subcore_barrier()

        pltpu.sync_copy(acc.at[pl.ds(row0, rpt)],
                        out_hbm.at[c, pl.ds(row0, rpt)])
        if tail:
            @pl.when(s == 0)
            def _():
                pltpu.sync_copy(acc.at[pl.ds(tail0, tail)],
                                out_hbm.at[c, pl.ds(tail0, tail)])

    return segsum


def _dense_body(agg0_ref, agg1_ref, w10, b10, w20, b20, w11, b11, w21, b21,
                sa_w, sa_b, sa_q, out_ref):
    a0_ref = agg0_ref.at[0]
    a1_ref = agg1_ref.at[0]
    def gin_mlp(a, w1, b1, w2, b2):
        h = jnp.maximum(
            jnp.dot(a, w1[...], preferred_element_type=jnp.float32) + b1[...], 0.0)
        t = jnp.dot(h, w2[...], preferred_element_type=jnp.float32) + b2[...]
        return jnp.where(t > 0, t, jnp.exp(jnp.minimum(t, 0.0)) - 1.0)

    e0 = gin_mlp(a0_ref[...], w10, b10, w20, b20)
    e1 = gin_mlp(a1_ref[...], w11, b11, w21, b21)

    def score(e):
        w = jnp.tanh(jnp.dot(e, sa_w[...], preferred_element_type=jnp.float32)
                     + sa_b[...])
        return jnp.dot(w, sa_q[...], preferred_element_type=jnp.float32)

    s0 = score(e0)
    s1 = score(e1)
    m = jnp.maximum(s0, s1)
    x0 = jnp.exp(s0 - m)
    x1 = jnp.exp(s1 - m)
    inv = 1.0 / (x0 + x1)
    out_ref[...] = (x0 * inv) * e0 + (x1 * inv) * e1


def _make_dense(n, feat, dim_a, block_rows=2000):
    assert n % block_rows == 0
    grid = n // block_rows
    row_spec = pl.BlockSpec((block_rows, feat), lambda i: (i, 0))
    agg_spec = lambda r: pl.BlockSpec((1, block_rows, feat), lambda i: (r, i, 0))
    full = lambda shape: pl.BlockSpec(shape, lambda i: (0,) * len(shape))
    return pl.pallas_call(
        _dense_body,
        grid=(grid,),
        in_specs=[
            agg_spec(0), agg_spec(1),
            full((feat, feat)), full((1, feat)), full((feat, feat)), full((1, feat)),
            full((feat, feat)), full((1, feat)), full((feat, feat)), full((1, feat)),
            full((feat, dim_a)), full((1, dim_a)), full((dim_a, 1)),
        ],
        out_specs=row_spec,
        out_shape=jax.ShapeDtypeStruct((n, feat), jnp.float32),
    )


def kernel(x, edge_index_r0, edge_index_r1,
           l0_r0_W1, l0_r0_b1, l0_r0_W2, l0_r0_b2,
           l0_r1_W1, l0_r1_b1, l0_r1_W2, l0_r1_b2,
           l1_r0_W1, l1_r0_b1, l1_r0_W2, l1_r0_b2,
           l1_r1_W1, l1_r1_b1, l1_r1_W2, l1_r1_b2,
           sa_W, sa_b, sa_q):
    n, feat = x.shape
    e = edge_index_r0.shape[1]
    dim_a = sa_W.shape[1]

    grain = N_TILES * CHUNK * SEG
    e_pad = -(-e // grain) * grain
    npad = e_pad - e
    pad_src = (jnp.arange(npad, dtype=jnp.int32) * 613) % n
    pad_dst = n + jnp.arange(npad, dtype=jnp.int32) % PAD_ROWS
    srcs = jnp.concatenate([edge_index_r0[0], pad_src,
                            edge_index_r1[0], pad_src]).reshape(-1, CHUNK)
    dsts = jnp.concatenate([edge_index_r0[1], pad_dst,
                            edge_index_r1[1], pad_dst]).reshape(-1, CHUNK)

    segsum = _make_segsum(n, feat, e_pad // CHUNK)
    dense = _make_dense(n, feat, dim_a)

    def layer(h, params):
        (w10, b10, w20, b20), (w11, b11, w21, b21) = params
        agg = segsum(h, srcs, dsts)
        return dense(agg, agg,
                     w10, b10.reshape(1, feat), w20, b20.reshape(1, feat),
                     w11, b11.reshape(1, feat), w21, b21.reshape(1, feat),
                     sa_W, sa_b, sa_q)

    h = layer(x, ((l0_r0_W1, l0_r0_b1, l0_r0_W2, l0_r0_b2),
                  (l0_r1_W1, l0_r1_b1, l0_r1_W2, l0_r1_b2)))
    h = layer(h, ((l1_r0_W1, l1_r0_b1, l1_r0_W2, l1_r0_b2),
                  (l1_r1_W1, l1_r1_b1, l1_r1_W2, l1_r1_b2)))
    return h
